# baseline (speedup 1.0000x reference)
import functools

import jax
import jax.numpy as jnp
from jax import lax
from jax.experimental import pallas as pl
from jax.experimental.pallas import tpu as pltpu
from jax.experimental.pallas import tpu_sc as plsc

HEAD_DIM = 128
HALF = HEAD_DIM // 2

_NUM_CORES = 2
_NUM_SUBCORES = 16
_NUM_WORKERS = _NUM_CORES * _NUM_SUBCORES
_CHUNK = 256
_NSLOT = 3
_LANES = 16


@functools.lru_cache(maxsize=None)
def _make_gather(batch, seq):
    n_rows = batch * seq
    b_per_w = n_rows // _NUM_WORKERS
    w_per_b = seq // b_per_w
    n_chunks = b_per_w // _CHUNK
    mesh = plsc.VectorSubcoreMesh(core_axis_name="c", subcore_axis_name="s")

    buf_types = [pltpu.VMEM((_CHUNK, HALF), jnp.float32)
                 for _ in range(2 * _NSLOT)]
    sem_types = [pltpu.SemaphoreType.DMA for _ in range(2 * _NSLOT)]

    @functools.partial(
        pl.kernel,
        mesh=mesh,
        out_type=[
            jax.ShapeDtypeStruct((batch, 1, seq, 2, HALF), jnp.float32),
            jax.ShapeDtypeStruct((batch, 1, seq, 2, HALF), jnp.float32),
        ],
        scratch_types=[pltpu.VMEM((b_per_w,), jnp.int32),
                       pltpu.VMEM((b_per_w,), jnp.int32)]
                      + buf_types + sem_types,
        compiler_params=pltpu.CompilerParams(use_tc_tiling_on_sc=False),
    )
    def gather_kernel(cos_hbm, sin_hbm, idx_hbm, cos_out, sin_out,
                      idx_s, idx_v, *bufs_and_sems):
        cbufs = bufs_and_sems[0:_NSLOT]
        sbufs = bufs_and_sems[_NSLOT:2 * _NSLOT]
        gsems = bufs_and_sems[2 * _NSLOT:3 * _NSLOT]
        wsems = bufs_and_sems[3 * _NSLOT:4 * _NSLOT]

        wid = lax.axis_index("s") * _NUM_CORES + lax.axis_index("c")
        bi = wid // w_per_b
        inner = (wid % w_per_b) * b_per_w
        pltpu.sync_copy(idx_hbm.at[bi, pl.ds(inner, b_per_w)], idx_s)

        def dbl(j, carry):
            sl = pl.ds(j * _LANES, _LANES)
            idx_v[sl] = idx_s[sl] * 2
            return carry

        lax.fori_loop(0, b_per_w // _LANES, dbl, 0)

        def issue_gather(i):
            s = i % _NSLOT
            sl = idx_v.at[pl.ds(i * _CHUNK, _CHUNK)]
            return (pltpu.async_copy(cos_hbm.at[sl], cbufs[s], gsems[s]),
                    pltpu.async_copy(sin_hbm.at[sl], sbufs[s], gsems[s]))

        def issue_write(i):
            s = i % _NSLOT
            rows = pl.ds(inner + i * _CHUNK, _CHUNK)
            return (pltpu.async_copy(cbufs[s], cos_out.at[bi, 0, rows, 0, :],
                                     wsems[s]),
                    pltpu.async_copy(cbufs[s], cos_out.at[bi, 0, rows, 1, :],
                                     wsems[s]),
                    pltpu.async_copy(sbufs[s], sin_out.at[bi, 0, rows, 0, :],
                                     wsems[s]),
                    pltpu.async_copy(sbufs[s], sin_out.at[bi, 0, rows, 1, :],
                                     wsems[s]))

        gh = {}
        wh = {}
        for i in range(min(2, n_chunks)):
            gh[i] = issue_gather(i)
        for i in range(n_chunks):
            if i >= 1:
                for h in wh.pop(i - 1):
                    h.wait()
            if i + 2 < n_chunks:
                gh[i + 2] = issue_gather(i + 2)
            for h in gh.pop(i):
                h.wait()
            wh[i] = issue_write(i)
        for h in wh.pop(n_chunks - 1):
            h.wait()

    return gather_kernel


def kernel(cos_cached, sin_cached, position_ids, batch_size, seq_len):
    del batch_size, seq_len
    b, s = position_ids.shape
    max_pos = cos_cached.shape[2]
    cos_half = cos_cached[0, 0].reshape(2 * max_pos, HALF)
    sin_half = sin_cached[0, 0].reshape(2 * max_pos, HALF)
    cos5, sin5 = _make_gather(b, s)(cos_half, sin_half, position_ids)
    cos = cos5.reshape(b, 1, s, HEAD_DIM)
    sin = sin5.reshape(b, 1, s, HEAD_DIM)
    return (cos, sin)
